# baseline (speedup 1.0000x reference)
import functools

import jax
import jax.numpy as jnp
from jax import lax
from jax.experimental import pallas as pl
from jax.experimental.pallas import tpu as pltpu
from jax.experimental.pallas import tpu_sc as plsc

NC, NS = 2, 16
NW = NC * NS
K = 64
CB = 32
RB = 512


DW = 128


def _deg_kernel(npad, nb):
    chunk = npad // NS
    nz = chunk // K
    mesh = plsc.VectorSubcoreMesh(core_axis_name="c", subcore_axis_name="s")

    @functools.partial(
        pl.kernel,
        out_type=jax.ShapeDtypeStruct((NC, npad, DW), jnp.float32),
        mesh=mesh,
        scratch_types=[
            pltpu.VMEM((nb, K), jnp.int32),
            pltpu.VMEM((K, DW), jnp.float32),
            pltpu.VMEM_SHARED((npad, DW), jnp.float32),
            pltpu.SemaphoreType.DMA,
            pltpu.SemaphoreType.DMA,
            pltpu.SemaphoreType.DMA,
            pltpu.SemaphoreType.DMA,
            pltpu.SemaphoreType.DMA,
            pltpu.SemaphoreType.DMA,
            pltpu.SemaphoreType.DMA,
            pltpu.SemaphoreType.DMA,
        ],
    )
    def deg(dst_hbm, zeros_hbm, ones_hbm, out_hbm, dst_v, rows_v, acc,
            d0, d1, d2, d3, d4, d5, d6, d7):
        c = lax.axis_index("c")
        s = lax.axis_index("s")
        pltpu.sync_copy(dst_hbm.at[c, s, pl.ds(0, nb // 2)],
                        dst_v.at[pl.ds(0, nb // 2)])
        pltpu.sync_copy(dst_hbm.at[c, s, pl.ds(nb // 2, nb // 2)],
                        dst_v.at[pl.ds(nb // 2, nb // 2)])
        pltpu.sync_copy(zeros_hbm, rows_v)
        for t in range(nz):
            pltpu.sync_copy(rows_v, acc.at[pl.ds(s * chunk + t * K, K)])
        plsc.subcore_barrier()
        pltpu.sync_copy(ones_hbm, rows_v)
        sems = (d0, d1, d2, d3, d4, d5, d6, d7)

        def body(i, carry):
            j = i * 8
            cps = [pltpu.async_copy(rows_v, acc.at[dst_v.at[j + t]], sems[t],
                                    add=True)
                   for t in range(8)]
            for t in range(8):
                cps[t].wait()
            return carry

        lax.fori_loop(0, nb // 8, body, 0)
        plsc.subcore_barrier()
        for t in range(nz):
            pltpu.sync_copy(acc.at[pl.ds(s * chunk + t * K, K)], rows_v)
            pltpu.sync_copy(rows_v, out_hbm.at[c, pl.ds(s * chunk + t * K, K)])

    return deg


def _scatter_kernel(npad, d, nb):
    chunk = npad // NS
    nz = chunk // K
    mesh = plsc.VectorSubcoreMesh(core_axis_name="c", subcore_axis_name="s")

    @functools.partial(
        pl.kernel,
        out_type=jax.ShapeDtypeStruct((NC, npad, d), jnp.float32),
        mesh=mesh,
        scratch_types=[
            pltpu.VMEM((CB, K), jnp.int32),
            pltpu.VMEM((4, K), jnp.int32),
            pltpu.VMEM((4, K), jnp.int32),
            pltpu.VMEM((K, d), jnp.float32),
            pltpu.VMEM((K, d), jnp.float32),
            pltpu.VMEM((K, d), jnp.float32),
            pltpu.VMEM_SHARED((npad, d), jnp.float32),
            pltpu.SemaphoreType.DMA,
            pltpu.SemaphoreType.DMA,
            pltpu.SemaphoreType.DMA,
            pltpu.SemaphoreType.DMA,
            pltpu.SemaphoreType.DMA,
            pltpu.SemaphoreType.DMA,
        ],
    )
    def scat(g_hbm, ei_hbm, zeros_hbm, out_hbm,
             pk_v, sidx, didx, r0, r1, r2, acc, s0, s1, s2, s3, s4, s5):
        c = lax.axis_index("c")
        s = lax.axis_index("s")
        pltpu.sync_copy(zeros_hbm, r0)
        for t in range(nz):
            pltpu.sync_copy(r0, acc.at[pl.ds(s * chunk + t * K, K)])
        plsc.subcore_barrier()

        def inner(i, o):
            j = i * 4
            for t in range(4):
                for u in range(K // 16):
                    v = pk_v[j + t, pl.ds(u * 16, 16)]
                    sidx[t, pl.ds(u * 16, 16)] = jnp.bitwise_and(v, 0xFFFF)
                    didx[t, pl.ds(u * 16, 16)] = jnp.right_shift(v, 16)
            cp0 = pltpu.async_copy(g_hbm.at[sidx.at[0]], r0, s0)
            cp1 = pltpu.async_copy(g_hbm.at[sidx.at[1]], r1, s1)
            cp2 = pltpu.async_copy(g_hbm.at[sidx.at[2]], r2, s2)
            cp0.wait()
            a0 = pltpu.async_copy(r0, acc.at[didx.at[0]], s3, add=True)
            a0.wait()
            cp3 = pltpu.async_copy(g_hbm.at[sidx.at[3]], r0, s0)
            cp1.wait()
            a1 = pltpu.async_copy(r1, acc.at[didx.at[1]], s4, add=True)
            cp2.wait()
            a2 = pltpu.async_copy(r2, acc.at[didx.at[2]], s5, add=True)
            cp3.wait()
            a3 = pltpu.async_copy(r0, acc.at[didx.at[3]], s3, add=True)
            a1.wait()
            a2.wait()
            a3.wait()
            return o

        def outer(o, carry):
            pltpu.sync_copy(ei_hbm.at[c, s, pl.ds(o * CB, CB)],
                            pk_v)
            lax.fori_loop(0, CB // 4, inner, o)
            return carry

        lax.fori_loop(0, nb // CB, outer, 0)
        plsc.subcore_barrier()
        for t in range(nz):
            pltpu.sync_copy(acc.at[pl.ds(s * chunk + t * K, K)], r0)
            pltpu.sync_copy(r0, out_hbm.at[c, pl.ds(s * chunk + t * K, K)])

    return scat


def _prescale_call(npad, d):
    def body(deg_ref, x_ref, g_ref, dinv_ref):
        degs = (deg_ref[0] + deg_ref[1])[:, :1]
        dinv = lax.rsqrt(degs + 1.0)
        dinv_ref[...] = dinv
        g_ref[...] = x_ref[...] * dinv

    return pl.pallas_call(
        body,
        grid=(npad // RB,),
        in_specs=[
            pl.BlockSpec((NC, RB, DW), lambda i: (0, i, 0)),
            pl.BlockSpec((RB, d), lambda i: (i, 0)),
        ],
        out_specs=[
            pl.BlockSpec((RB, d), lambda i: (i, 0)),
            pl.BlockSpec((RB, 1), lambda i: (i, 0)),
        ],
        out_shape=[
            jax.ShapeDtypeStruct((npad, d), jnp.float32),
            jax.ShapeDtypeStruct((npad, 1), jnp.float32),
        ],
    )


def _mid_call(npad, d, dh):
    def body(p_ref, g1_ref, dinv_ref, w1_ref, b1_ref, w2_ref, g2_ref):
        p = p_ref[0] + p_ref[1] + g1_ref[...]
        a = p * dinv_ref[...]
        h = jnp.dot(a, w1_ref[...], preferred_element_type=jnp.float32)
        h = jnp.maximum(h + b1_ref[...], 0.0)
        g2 = jnp.dot(h, w2_ref[...], preferred_element_type=jnp.float32)
        g2_ref[...] = g2 * dinv_ref[...]

    return pl.pallas_call(
        body,
        grid=(npad // RB,),
        in_specs=[
            pl.BlockSpec((NC, RB, d), lambda i: (0, i, 0)),
            pl.BlockSpec((RB, d), lambda i: (i, 0)),
            pl.BlockSpec((RB, 1), lambda i: (i, 0)),
            pl.BlockSpec((d, dh), lambda i: (0, 0)),
            pl.BlockSpec((1, dh), lambda i: (0, 0)),
            pl.BlockSpec((dh, d), lambda i: (0, 0)),
        ],
        out_specs=pl.BlockSpec((RB, d), lambda i: (i, 0)),
        out_shape=jax.ShapeDtypeStruct((npad, d), jnp.float32),
    )


def _out_call(n, d):
    r = 400

    def body(p_ref, g2_ref, dinv_ref, b2_ref, o_ref):
        t = (p_ref[0] + p_ref[1] + g2_ref[...]) * dinv_ref[...] + b2_ref[...]
        o_ref[...] = jnp.maximum(t, 0.0)

    return pl.pallas_call(
        body,
        grid=(n // r,),
        in_specs=[
            pl.BlockSpec((NC, r, d), lambda i: (0, i, 0)),
            pl.BlockSpec((r, d), lambda i: (i, 0)),
            pl.BlockSpec((r, 1), lambda i: (i, 0)),
            pl.BlockSpec((1, d), lambda i: (0, 0)),
        ],
        out_specs=pl.BlockSpec((r, d), lambda i: (i, 0)),
        out_shape=jax.ShapeDtypeStruct((n, d), jnp.float32),
    )


def kernel(x, edge_index, W1, b1, W2, b2):
    n, d = x.shape
    dh = W1.shape[1]
    e = edge_index.shape[1]
    npad = ((n + 1 + NS * K - 1) // (NS * K)) * (NS * K)
    nb = (e + NW * K - 1) // (NW * K)
    nb = (nb + CB - 1) // CB * CB
    pad_e = NW * nb * K - e

    src = edge_index[0].astype(jnp.int32)
    dst = edge_index[1].astype(jnp.int32)
    fill = n + (jnp.arange(pad_e, dtype=jnp.int32) % (npad - n))
    srcf = jnp.concatenate([src, fill])
    dstf = jnp.concatenate([dst, fill])
    pk = jnp.bitwise_or(srcf, jnp.left_shift(dstf, 16)).reshape(NC, NS, nb, K)
    dstp = dstf.reshape(NC, NS, nb, K)
    x_pad = jnp.concatenate([x, jnp.zeros((npad - n, d), x.dtype)])
    zerosd = jnp.zeros((K, d), jnp.float32)
    zerosw = jnp.zeros((K, DW), jnp.float32)
    onesw = jnp.ones((K, DW), jnp.float32)

    deg = _deg_kernel(npad, nb)(dstp, zerosw, onesw)
    g1, dinv = _prescale_call(npad, d)(deg, x_pad)
    parts1 = _scatter_kernel(npad, d, nb)(g1, pk, zerosd)
    g2 = _mid_call(npad, d, dh)(parts1, g1, dinv, W1, b1.reshape(1, dh), W2)
    parts2 = _scatter_kernel(npad, d, nb)(g2, pk, zerosd)
    out = _out_call(n, d)(parts2, g2, dinv, b2.reshape(1, d))
    return out
